# baseline (speedup 1.0000x reference)
import jax
import jax.numpy as jnp
from jax import lax
from jax.experimental import pallas as pl
from jax.experimental.pallas import tpu as pltpu

SEQ = 2048
D_MODEL = 768
N_EXPERTS = 8
TOP_K = 2
D_FF = 4 * D_MODEL

NPAIR = SEQ * TOP_K
TM = 256
NP_ROWS = NPAIR + N_EXPERTS * TM
NT = NP_ROWS // TM


def _gate_route_body(x_ref, wg_ref, bg_ref, wts_ref, dest_ref, meta_ref,
                     xb_ref):
    x = x_ref[...]
    xb_ref[...] = x.astype(jnp.bfloat16)
    logits = jnp.dot(x, wg_ref[...], preferred_element_type=jnp.float32)
    logits = logits + bg_ref[...]
    iota = lax.broadcasted_iota(jnp.int32, logits.shape, 1)
    m1 = jnp.max(logits, axis=1, keepdims=True)
    i1 = jnp.min(jnp.where(logits >= m1, iota, N_EXPERTS), axis=1, keepdims=True)
    lmask = jnp.where(iota == i1, -jnp.inf, logits)
    m2 = jnp.max(lmask, axis=1, keepdims=True)
    i2 = jnp.min(jnp.where(lmask >= m2, iota, N_EXPERTS), axis=1, keepdims=True)
    z = jnp.sum(jnp.exp(logits - m1), axis=1, keepdims=True)
    wts_ref[...] = jnp.concatenate([1.0 / z, jnp.exp(m2 - m1) / z], axis=1)

    iotaE1 = lax.broadcasted_iota(jnp.int32, (SEQ, N_EXPERTS), 1)
    oh = jnp.concatenate([(iotaE1 == i1), (iotaE1 == i2)], axis=0)
    oh = oh.astype(jnp.int32)
    s = oh
    d = 1
    while d < NPAIR:
        s = s + jnp.concatenate(
            [jnp.zeros((d, N_EXPERTS), jnp.int32), s[: NPAIR - d, :]], axis=0)
        d *= 2
    rank = jnp.sum(oh * s, axis=1, keepdims=True)
    counts = s[NPAIR - 1 :, :]
    cpad = ((counts + TM - 1) // TM) * TM
    e = cpad
    d = 1
    while d < N_EXPERTS:
        e = e + jnp.concatenate(
            [jnp.zeros((1, d), jnp.int32), e[:, : N_EXPERTS - d]], axis=1)
        d *= 2
    ends = e
    starts = ends - cpad
    start_pp = jnp.sum(oh * starts, axis=1, keepdims=True)
    dest_ref[...] = start_pp + rank - 1

    tvec = lax.broadcasted_iota(jnp.int32, (NT, N_EXPERTS), 0) * TM
    te = jnp.minimum(jnp.sum((tvec >= ends).astype(jnp.int32), axis=1,
                             keepdims=True), N_EXPERTS - 1)
    valid = (tvec[:, :1] < ends[:, N_EXPERTS - 1 :]).astype(jnp.int32)
    meta_ref[...] = jnp.concatenate([te, valid], axis=0)


def _gate_route(xt, Wg, bg):
    return pl.pallas_call(
        _gate_route_body,
        out_shape=(
            jax.ShapeDtypeStruct((SEQ, TOP_K), jnp.float32),
            jax.ShapeDtypeStruct((NPAIR, 1), jnp.int32),
            jax.ShapeDtypeStruct((2 * NT, 1), jnp.int32),
            jax.ShapeDtypeStruct((SEQ, D_MODEL), jnp.bfloat16),
        ),
    )(xt, Wg, bg.reshape(1, N_EXPERTS))


def _moe_body(meta_ref, x_ref, w1_ref, b1_ref, w2_ref, b2_ref, d2_ref,
              wts_ref, out_ref):
    t = pl.program_id(0)
    base = t * TM
    valid = meta_ref[NT + t] == 1

    @pl.when(valid)
    def _():
        d0r = d2_ref[0, :] - base
        d1r = d2_ref[1, :] - base
        iota_r = lax.broadcasted_iota(jnp.int32, (TM, SEQ), 0)
        hit = jnp.logical_or(d0r[None, :] == iota_r, d1r[None, :] == iota_r)
        p = hit.astype(jnp.bfloat16)
        xs = jnp.dot(p, x_ref[...], preferred_element_type=jnp.float32)
        xs = xs.astype(jnp.bfloat16)
        w1 = w1_ref[0].astype(jnp.bfloat16)
        h = jnp.dot(xs, w1, preferred_element_type=jnp.float32)
        h = jnp.maximum(h + b1_ref[0, 0][None, :], 0.0).astype(jnp.bfloat16)
        w2 = w2_ref[0].astype(jnp.bfloat16)
        y = jnp.dot(h, w2, preferred_element_type=jnp.float32)
        y = y + b2_ref[0, 0][None, :]
        w = wts_ref[...].astype(jnp.bfloat16)
        iota_c = lax.broadcasted_iota(jnp.int32, (SEQ, TM), 1)
        eq0 = (d0r[:, None] == iota_c).astype(jnp.bfloat16)
        eq1 = (d1r[:, None] == iota_c).astype(jnp.bfloat16)
        gt = eq0 * w[:, :1] + eq1 * w[:, 1:2]
        contrib = jnp.dot(gt, y.astype(jnp.bfloat16),
                          preferred_element_type=jnp.float32)

        @pl.when(t == 0)
        def _():
            out_ref[...] = contrib

        @pl.when(t > 0)
        def _():
            out_ref[...] += contrib


def _moe(meta, xb, W1, b1, W2, b2, dest2, wts):
    grid_spec = pltpu.PrefetchScalarGridSpec(
        num_scalar_prefetch=1,
        grid=(NT,),
        in_specs=[
            pl.BlockSpec((SEQ, D_MODEL), lambda t, m: (0, 0)),
            pl.BlockSpec((1, D_MODEL, D_FF), lambda t, m: (m[t], 0, 0)),
            pl.BlockSpec((1, 1, D_FF), lambda t, m: (m[t], 0, 0)),
            pl.BlockSpec((1, D_FF, D_MODEL), lambda t, m: (m[t], 0, 0)),
            pl.BlockSpec((1, 1, D_MODEL), lambda t, m: (m[t], 0, 0)),
            pl.BlockSpec((2, SEQ), lambda t, m: (0, 0)),
            pl.BlockSpec((SEQ, TOP_K), lambda t, m: (0, 0)),
        ],
        out_specs=pl.BlockSpec((SEQ, D_MODEL), lambda t, m: (0, 0)),
    )
    return pl.pallas_call(
        _moe_body,
        grid_spec=grid_spec,
        out_shape=jax.ShapeDtypeStruct((SEQ, D_MODEL), jnp.float32),
    )(
        meta,
        xb,
        W1,
        b1.reshape(N_EXPERTS, 1, D_FF),
        W2,
        b2.reshape(N_EXPERTS, 1, D_MODEL),
        dest2,
        wts,
    )


def kernel(x, Wg, bg, W1, b1, W2, b2):
    B, S, D = x.shape
    xt = x.reshape(S, D)

    wts, dest, meta2, xb = _gate_route(xt, Wg, bg)
    meta = meta2.reshape(2 * NT)
    dest2 = dest.reshape(TOP_K, SEQ)

    out = _moe(meta, xb, W1, b1, W2, b2, dest2, wts)
    return out.reshape(B, S, D)
